# baseline (speedup 1.0000x reference)
import numpy as np
import jax
import jax.numpy as jnp
from jax.experimental import pallas as pl

_FROM = (0, 1, 2, 3, 4, 5, 6, 3, 8, 9, 10, 3, 12, 13, 14, 0, 16, 17, 18, 0, 20, 21)
_TO = tuple(range(1, 23))
_NUM_LIMBS = 22
_BPR = 2
_NF = 69
_ROW = _BPR * _NF
_G = _BPR * _NUM_LIMBS


def _selection_matrix() -> np.ndarray:
    sel = np.zeros((_ROW, 3 * _G), dtype=np.float32)
    for j in range(_BPR):
        for c in range(3):
            for l in range(_NUM_LIMBS):
                col = c * _G + j * _NUM_LIMBS + l
                sel[j * _NF + c * 23 + _FROM[l], col] += 1.0
                sel[j * _NF + c * 23 + _TO[l], col] -= 1.0
    return sel


def _loss_kernel(gt_ref, pr_ref, sel_ref, out_ref):
    i = pl.program_id(0)
    d = gt_ref[...] - pr_ref[...]
    y = jnp.dot(d, sel_ref[...], preferred_element_type=jnp.float32)
    sq = y * y
    v = sq[:, 0:_G] + sq[:, _G : 2 * _G] + sq[:, 2 * _G : 3 * _G]
    part = jnp.sum(jnp.sqrt(v)).reshape(1, 1)

    @pl.when(i == 0)
    def _():
        out_ref[...] = jnp.zeros((1, 1), jnp.float32)

    out_ref[...] += part


def kernel(kpts_gt, kpts_pred):
    n, ncoord, nkpt = kpts_gt.shape
    nrows = n // _BPR
    block_r = 2048
    grid = nrows // block_r
    sel = jnp.asarray(_selection_matrix())
    gt2 = kpts_gt.reshape(nrows, _ROW)
    pr2 = kpts_pred.reshape(nrows, _ROW)
    total = pl.pallas_call(
        _loss_kernel,
        grid=(grid,),
        in_specs=[
            pl.BlockSpec((block_r, _ROW), lambda i: (i, 0)),
            pl.BlockSpec((block_r, _ROW), lambda i: (i, 0)),
            pl.BlockSpec((_ROW, 3 * _G), lambda i: (0, 0)),
        ],
        out_specs=pl.BlockSpec((1, 1), lambda i: (0, 0)),
        out_shape=jax.ShapeDtypeStruct((1, 1), jnp.float32),
    )(gt2, pr2, sel)
    return total[0, 0] / np.float32(n * _NUM_LIMBS)


# speedup vs baseline: 4.4710x; 4.4710x over previous
import numpy as np
import jax
import jax.numpy as jnp
from jax.experimental import pallas as pl

_FROM = (0, 1, 2, 3, 4, 5, 6, 3, 8, 9, 10, 3, 12, 13, 14, 0, 16, 17, 18, 0, 20, 21)
_TO = tuple(range(1, 23))
_NUM_LIMBS = 22


def _selection_matrix() -> np.ndarray:
    sel = np.zeros((69, 128), dtype=np.float32)
    for c in range(3):
        for l in range(_NUM_LIMBS):
            sel[c * 23 + _FROM[l], 32 * c + l] += 1.0
            sel[c * 23 + _TO[l], 32 * c + l] -= 1.0
    return sel


def _loss_kernel(gt_ref, pr_ref, sel_ref, out_ref):
    i = pl.program_id(0)
    d = gt_ref[...] - pr_ref[...]
    y = jnp.dot(d, sel_ref[...], preferred_element_type=jnp.float32)
    sq = y * y
    v = sq[:, 0:32] + sq[:, 32:64] + sq[:, 64:96]
    part = jnp.sum(jnp.sqrt(v)).reshape(1, 1)

    @pl.when(i == 0)
    def _():
        out_ref[...] = jnp.zeros((1, 1), jnp.float32)

    out_ref[...] += part


def kernel(kpts_gt, kpts_pred):
    n, ncoord, nkpt = kpts_gt.shape
    nfeat = ncoord * nkpt
    block_b = 4096
    grid = n // block_b
    sel = jnp.asarray(_selection_matrix())
    gt2 = kpts_gt.reshape(n, nfeat)
    pr2 = kpts_pred.reshape(n, nfeat)
    total = pl.pallas_call(
        _loss_kernel,
        grid=(grid,),
        in_specs=[
            pl.BlockSpec((block_b, nfeat), lambda i: (i, 0)),
            pl.BlockSpec((block_b, nfeat), lambda i: (i, 0)),
            pl.BlockSpec((nfeat, 128), lambda i: (0, 0)),
        ],
        out_specs=pl.BlockSpec((1, 1), lambda i: (0, 0)),
        out_shape=jax.ShapeDtypeStruct((1, 1), jnp.float32),
    )(gt2, pr2, sel)
    return total[0, 0] / np.float32(n * _NUM_LIMBS)
